# baseline (speedup 1.0000x reference)
import functools

import jax
import jax.numpy as jnp
from jax import lax
from jax.experimental import pallas as pl
from jax.experimental.pallas import tpu as pltpu
from jax.experimental.pallas import tpu_sc as plsc

B, N1, N2, C1, C2 = 4, 4096, 1024, 128, 256
H1, H2 = 256, 128
IN_CH = C1 + C2
BQ = 1024
BQA = 2048
BQ2 = 2048
NTOT = B * N1
EPS_D = 1e-10
EPS_BN = 1e-5

NW = 32
QPW = (B * N1) // NW
CH = 64
NCH = QPW // CH


def _recip(x):
    r = 1.0 / x
    r = r * (2.0 - x * r)
    r = r * (2.0 - x * r)
    return r


def _rsqrt(x):
    y = jax.lax.rsqrt(x)
    y = y * (1.5 - 0.5 * x * y * y)
    y = y * (1.5 - 0.5 * x * y * y)
    return y


def _stageA_kernel(xyz1t_ref, xyz2t_ref, idx_ref, w_ref):
    b = pl.program_id(0)

    q = xyz1t_ref[0]
    p = xyz2t_ref[0]
    d = (p[0][:, None] - q[0][None, :]) ** 2
    d = d + (p[1][:, None] - q[1][None, :]) ** 2
    d = d + (p[2][:, None] - q[2][None, :]) ** 2
    d = jnp.maximum(d, EPS_D)
    m1 = jnp.min(d, axis=0)
    d1 = jnp.where(d <= m1[None, :], jnp.inf, d)
    m2 = jnp.min(d1, axis=0)
    d2 = jnp.where(d1 <= m2[None, :], jnp.inf, d1)
    m3 = jnp.min(d2, axis=0)
    iota = lax.broadcasted_iota(jnp.int32, (N2, BQ), 0)
    j1 = jnp.min(jnp.where(d == m1[None, :], iota, N2), axis=0)
    j2 = jnp.min(jnp.where(d == m2[None, :], iota, N2), axis=0)
    j3 = jnp.min(jnp.where(d == m3[None, :], iota, N2), axis=0)
    idx_ref[...] = jnp.stack([j1, j2, j3], axis=0) + b * N2
    r1 = _recip(m1)
    r2 = _recip(m2)
    r3 = _recip(m3)
    rs = _recip(r1 + r2 + r3)
    w_ref[...] = jnp.stack([r1 * rs, r2 * rs, r3 * rs], axis=1)


def _sc_gather_kernel(table_ref, idx_ref, w_ref, out_ref,
                      idx0_v, idx1_v, idx2_v, w_v, rows_v, out_v, sem):
    cid = lax.axis_index("c")
    sid = lax.axis_index("s")
    wid = sid * 2 + cid
    base = wid * QPW

    def chunk_body(ch, carry):
        off = base + ch * CH
        pltpu.sync_copy(idx_ref.at[pl.ds(off, CH)], idx0_v)
        pltpu.sync_copy(idx_ref.at[pl.ds(NTOT + off, CH)], idx1_v)
        pltpu.sync_copy(idx_ref.at[pl.ds(2 * NTOT + off, CH)], idx2_v)
        pltpu.sync_copy(w_ref.at[pl.ds(off * 3, 3 * CH)],
                        w_v.at[pl.ds(0, 3 * CH)])
        for k, idxk_v in enumerate((idx0_v, idx1_v, idx2_v)):
            pltpu.async_copy(table_ref.at[idxk_v], rows_v.at[k],
                             sem).wait()

        def q_body(qi, c2):
            wv = w_v[pl.ds(qi * 3, 16)]
            w0 = wv[0]
            w1 = wv[1]
            w2 = wv[2]
            for cc in range(C2 // 16):
                sl = pl.ds(cc * 16, 16)
                v = (w0 * rows_v[0, qi, sl] + w1 * rows_v[1, qi, sl]
                     + w2 * rows_v[2, qi, sl])
                out_v[qi, sl] = v
            return c2

        lax.fori_loop(0, CH, q_body, 0)
        pltpu.sync_copy(out_v, out_ref.at[pl.ds(off, CH)])
        return carry

    lax.fori_loop(0, NCH, chunk_body, 0)


def _stageAp_kernel(p1_ref, interp_ref, W1a_ref, W1b_ref, b1_ref,
                    y1_ref, ssum_ref, ssq_ref):
    b = pl.program_id(0)
    i = pl.program_id(1)

    @pl.when(jnp.logical_and(b == 0, i == 0))
    def _init():
        ssum_ref[...] = jnp.zeros_like(ssum_ref)
        ssq_ref[...] = jnp.zeros_like(ssq_ref)

    y1 = jnp.dot(W1a_ref[...], p1_ref[0], preferred_element_type=jnp.float32)
    y1 = y1 + lax.dot_general(W1b_ref[...], interp_ref[0],
                              (((1,), (1,)), ((), ())),
                              preferred_element_type=jnp.float32)
    y1 = y1 + b1_ref[...]
    y1_ref[0] = y1
    ssum_ref[...] += jnp.sum(y1, axis=1, keepdims=True)
    ssq_ref[...] += jnp.sum(y1 * y1, axis=1, keepdims=True)


def _stage2_kernel(y1_ref, W2_ref, g1_ref, be1_ref, b2_ref, s_ref, sq_ref,
                   y2_ref, ssum_ref, ssq_ref):
    b = pl.program_id(0)
    i = pl.program_id(1)

    @pl.when(jnp.logical_and(b == 0, i == 0))
    def _init():
        ssum_ref[...] = jnp.zeros_like(ssum_ref)
        ssq_ref[...] = jnp.zeros_like(ssq_ref)

    mean = s_ref[...] * (1.0 / NTOT)
    var = sq_ref[...] * (1.0 / NTOT) - mean * mean
    scale = g1_ref[...] * _rsqrt(var + EPS_BN)
    h = (y1_ref[0] - mean) * scale + be1_ref[...]
    h = jnp.maximum(h, 0.0)
    y2 = jnp.dot(W2_ref[...], h, preferred_element_type=jnp.float32)
    y2 = y2 + b2_ref[...]
    y2_ref[0] = y2
    ssum_ref[...] += jnp.sum(y2, axis=1, keepdims=True)
    ssq_ref[...] += jnp.sum(y2 * y2, axis=1, keepdims=True)


def _stage3_kernel(y2_ref, g2_ref, be2_ref, s_ref, sq_ref, out_ref):
    mean = s_ref[...] * (1.0 / NTOT)
    var = sq_ref[...] * (1.0 / NTOT) - mean * mean
    scale = g2_ref[...] * _rsqrt(var + EPS_BN)
    o = (y2_ref[0] - mean) * scale + be2_ref[...]
    out_ref[0] = jnp.maximum(o, 0.0)


def _fp_impl(xyz1, xyz2, points1, points2, W1, b1, g1, be1, W2, b2, g2, be2,
             interpret=False):
    xyz1t = jnp.transpose(xyz1, (0, 2, 1))
    xyz2t = jnp.transpose(xyz2, (0, 2, 1))
    table = jnp.transpose(points2, (0, 2, 1)).reshape(B * N2, C2)
    W1a = W1[:, :C1]
    W1b = W1[:, C1:]
    b1c = b1[:, None]
    g1c = g1[:, None]
    be1c = be1[:, None]
    b2c = b2[:, None]
    g2c = g2[:, None]
    be2c = be2[:, None]

    f32 = jnp.float32
    idxg, wt = pl.pallas_call(
        _stageA_kernel,
        grid=(B, N1 // BQ),
        in_specs=[
            pl.BlockSpec((1, 3, BQ), lambda b, i: (b, 0, i)),
            pl.BlockSpec((1, 3, N2), lambda b, i: (b, 0, 0)),
        ],
        out_specs=[
            pl.BlockSpec((3, BQ), lambda b, i: (0, b * (N1 // BQ) + i)),
            pl.BlockSpec((BQ, 3), lambda b, i: (b * (N1 // BQ) + i, 0)),
        ],
        out_shape=[
            jax.ShapeDtypeStruct((3, B * N1), jnp.int32),
            jax.ShapeDtypeStruct((B * N1, 3), f32),
        ],
        interpret=interpret,
    )(xyz1t, xyz2t)

    sc_gather = functools.partial(
        pl.kernel,
        _sc_gather_kernel,
        out_type=jax.ShapeDtypeStruct((B * N1, C2), f32),
        mesh=plsc.VectorSubcoreMesh(core_axis_name="c", subcore_axis_name="s"),
        scratch_types=[
            pltpu.VMEM((CH,), jnp.int32),
            pltpu.VMEM((CH,), jnp.int32),
            pltpu.VMEM((CH,), jnp.int32),
            pltpu.VMEM((3 * CH + 16,), f32),
            pltpu.VMEM((3, CH, C2), f32),
            pltpu.VMEM((CH, C2), f32),
            pltpu.SemaphoreType.DMA,
        ],
    )
    interp = sc_gather()(table, idxg.reshape(3 * B * N1),
                         wt.reshape(B * N1 * 3))
    interp = interp.reshape(B, N1, C2)

    y1, s1, q1 = pl.pallas_call(
        _stageAp_kernel,
        grid=(B, N1 // BQA),
        in_specs=[
            pl.BlockSpec((1, C1, BQA), lambda b, i: (b, 0, i)),
            pl.BlockSpec((1, BQA, C2), lambda b, i: (b, i, 0)),
            pl.BlockSpec((H1, C1), lambda b, i: (0, 0)),
            pl.BlockSpec((H1, C2), lambda b, i: (0, 0)),
            pl.BlockSpec((H1, 1), lambda b, i: (0, 0)),
        ],
        out_specs=[
            pl.BlockSpec((1, H1, BQA), lambda b, i: (b, 0, i)),
            pl.BlockSpec((H1, 1), lambda b, i: (0, 0)),
            pl.BlockSpec((H1, 1), lambda b, i: (0, 0)),
        ],
        out_shape=[
            jax.ShapeDtypeStruct((B, H1, N1), f32),
            jax.ShapeDtypeStruct((H1, 1), f32),
            jax.ShapeDtypeStruct((H1, 1), f32),
        ],
        interpret=interpret,
    )(points1, interp, W1a, W1b, b1c)

    y2, s2, q2 = pl.pallas_call(
        _stage2_kernel,
        grid=(B, N1 // BQ2),
        in_specs=[
            pl.BlockSpec((1, H1, BQ2), lambda b, i: (b, 0, i)),
            pl.BlockSpec((H2, H1), lambda b, i: (0, 0)),
            pl.BlockSpec((H1, 1), lambda b, i: (0, 0)),
            pl.BlockSpec((H1, 1), lambda b, i: (0, 0)),
            pl.BlockSpec((H2, 1), lambda b, i: (0, 0)),
            pl.BlockSpec((H1, 1), lambda b, i: (0, 0)),
            pl.BlockSpec((H1, 1), lambda b, i: (0, 0)),
        ],
        out_specs=[
            pl.BlockSpec((1, H2, BQ2), lambda b, i: (b, 0, i)),
            pl.BlockSpec((H2, 1), lambda b, i: (0, 0)),
            pl.BlockSpec((H2, 1), lambda b, i: (0, 0)),
        ],
        out_shape=[
            jax.ShapeDtypeStruct((B, H2, N1), f32),
            jax.ShapeDtypeStruct((H2, 1), f32),
            jax.ShapeDtypeStruct((H2, 1), f32),
        ],
        interpret=interpret,
    )(y1, W2, g1c, be1c, b2c, s1, q1)

    out = pl.pallas_call(
        _stage3_kernel,
        grid=(B, N1 // BQ2),
        in_specs=[
            pl.BlockSpec((1, H2, BQ2), lambda b, i: (b, 0, i)),
            pl.BlockSpec((H2, 1), lambda b, i: (0, 0)),
            pl.BlockSpec((H2, 1), lambda b, i: (0, 0)),
            pl.BlockSpec((H2, 1), lambda b, i: (0, 0)),
            pl.BlockSpec((H2, 1), lambda b, i: (0, 0)),
        ],
        out_specs=pl.BlockSpec((1, H2, BQ2), lambda b, i: (b, 0, i)),
        out_shape=jax.ShapeDtypeStruct((B, H2, N1), f32),
        interpret=interpret,
    )(y2, g2c, be2c, s2, q2)
    return out


def kernel(xyz1, xyz2, points1, points2, W1, b1, g1, be1, W2, b2, g2, be2):
    return _fp_impl(xyz1, xyz2, points1, points2, W1, b1, g1, be1,
                    W2, b2, g2, be2)


# speedup vs baseline: 2.2212x; 2.2212x over previous
import jax
import jax.numpy as jnp
from jax.experimental import pallas as pl

B, N1, N2, C1, C2 = 4, 4096, 1024, 128, 256
H1, H2 = 256, 128
IN_CH = C1 + C2
BQ = 1024
BQ2 = 2048
NTOT = B * N1
EPS_D = 1e-10
EPS_BN = 1e-5


def _recip(x):
    r = 1.0 / x
    r = r * (2.0 - x * r)
    r = r * (2.0 - x * r)
    return r


def _rsqrt(x):
    y = jax.lax.rsqrt(x)
    y = y * (1.5 - 0.5 * x * y * y)
    y = y * (1.5 - 0.5 * x * y * y)
    return y


def _stage1_kernel(xyz1t_ref, xyz2t_ref, p1_ref, p2_ref, W1_ref, b1_ref,
                   y1_ref, ssum_ref, ssq_ref):
    b = pl.program_id(0)
    i = pl.program_id(1)

    @pl.when(jnp.logical_and(b == 0, i == 0))
    def _init():
        ssum_ref[...] = jnp.zeros_like(ssum_ref)
        ssq_ref[...] = jnp.zeros_like(ssq_ref)

    q = xyz1t_ref[0]
    p = xyz2t_ref[0]
    d = (p[0][:, None] - q[0][None, :]) ** 2
    d = d + (p[1][:, None] - q[1][None, :]) ** 2
    d = d + (p[2][:, None] - q[2][None, :]) ** 2
    d = jnp.maximum(d, EPS_D)
    m1 = jnp.min(d, axis=0)
    d1 = jnp.where(d <= m1[None, :], jnp.inf, d)
    m2 = jnp.min(d1, axis=0)
    d2 = jnp.where(d1 <= m2[None, :], jnp.inf, d1)
    m3 = jnp.min(d2, axis=0)
    w = jnp.where(d <= m3[None, :], 1.0 / d, 0.0)
    wsum = jnp.sum(w, axis=0)
    interp = jnp.dot(p2_ref[0], w, preferred_element_type=jnp.float32)
    interp = interp * _recip(wsum)[None, :]
    x = jnp.concatenate([p1_ref[0], interp], axis=0)
    y1 = jnp.dot(W1_ref[...], x, preferred_element_type=jnp.float32)
    y1 = y1 + b1_ref[...]
    y1_ref[0] = y1
    ssum_ref[...] += jnp.sum(y1, axis=1, keepdims=True)
    ssq_ref[...] += jnp.sum(y1 * y1, axis=1, keepdims=True)


def _stage2_kernel(y1_ref, W2_ref, g1_ref, be1_ref, b2_ref, s_ref, sq_ref,
                   y2_ref, ssum_ref, ssq_ref):
    b = pl.program_id(0)
    i = pl.program_id(1)

    @pl.when(jnp.logical_and(b == 0, i == 0))
    def _init():
        ssum_ref[...] = jnp.zeros_like(ssum_ref)
        ssq_ref[...] = jnp.zeros_like(ssq_ref)

    mean = s_ref[...] * (1.0 / NTOT)
    var = sq_ref[...] * (1.0 / NTOT) - mean * mean
    scale = g1_ref[...] * _rsqrt(var + EPS_BN)
    h = (y1_ref[0] - mean) * scale + be1_ref[...]
    h = jnp.maximum(h, 0.0)
    y2 = jnp.dot(W2_ref[...], h, preferred_element_type=jnp.float32)
    y2 = y2 + b2_ref[...]
    y2_ref[0] = y2
    ssum_ref[...] += jnp.sum(y2, axis=1, keepdims=True)
    ssq_ref[...] += jnp.sum(y2 * y2, axis=1, keepdims=True)


def _stage3_kernel(y2_ref, g2_ref, be2_ref, s_ref, sq_ref, out_ref):
    mean = s_ref[...] * (1.0 / NTOT)
    var = sq_ref[...] * (1.0 / NTOT) - mean * mean
    scale = g2_ref[...] * _rsqrt(var + EPS_BN)
    o = (y2_ref[0] - mean) * scale + be2_ref[...]
    out_ref[0] = jnp.maximum(o, 0.0)


def _fp_impl(xyz1, xyz2, points1, points2, W1, b1, g1, be1, W2, b2, g2, be2,
             interpret=False):
    xyz1t = jnp.transpose(xyz1, (0, 2, 1))
    xyz2t = jnp.transpose(xyz2, (0, 2, 1))
    b1c = b1[:, None]
    g1c = g1[:, None]
    be1c = be1[:, None]
    b2c = b2[:, None]
    g2c = g2[:, None]
    be2c = be2[:, None]

    f32 = jnp.float32
    y1, s1, q1 = pl.pallas_call(
        _stage1_kernel,
        grid=(B, N1 // BQ),
        in_specs=[
            pl.BlockSpec((1, 3, BQ), lambda b, i: (b, 0, i)),
            pl.BlockSpec((1, 3, N2), lambda b, i: (b, 0, 0)),
            pl.BlockSpec((1, C1, BQ), lambda b, i: (b, 0, i)),
            pl.BlockSpec((1, C2, N2), lambda b, i: (b, 0, 0)),
            pl.BlockSpec((H1, IN_CH), lambda b, i: (0, 0)),
            pl.BlockSpec((H1, 1), lambda b, i: (0, 0)),
        ],
        out_specs=[
            pl.BlockSpec((1, H1, BQ), lambda b, i: (b, 0, i)),
            pl.BlockSpec((H1, 1), lambda b, i: (0, 0)),
            pl.BlockSpec((H1, 1), lambda b, i: (0, 0)),
        ],
        out_shape=[
            jax.ShapeDtypeStruct((B, H1, N1), f32),
            jax.ShapeDtypeStruct((H1, 1), f32),
            jax.ShapeDtypeStruct((H1, 1), f32),
        ],
        interpret=interpret,
    )(xyz1t, xyz2t, points1, points2, W1, b1c)

    y2, s2, q2 = pl.pallas_call(
        _stage2_kernel,
        grid=(B, N1 // BQ2),
        in_specs=[
            pl.BlockSpec((1, H1, BQ2), lambda b, i: (b, 0, i)),
            pl.BlockSpec((H2, H1), lambda b, i: (0, 0)),
            pl.BlockSpec((H1, 1), lambda b, i: (0, 0)),
            pl.BlockSpec((H1, 1), lambda b, i: (0, 0)),
            pl.BlockSpec((H2, 1), lambda b, i: (0, 0)),
            pl.BlockSpec((H1, 1), lambda b, i: (0, 0)),
            pl.BlockSpec((H1, 1), lambda b, i: (0, 0)),
        ],
        out_specs=[
            pl.BlockSpec((1, H2, BQ2), lambda b, i: (b, 0, i)),
            pl.BlockSpec((H2, 1), lambda b, i: (0, 0)),
            pl.BlockSpec((H2, 1), lambda b, i: (0, 0)),
        ],
        out_shape=[
            jax.ShapeDtypeStruct((B, H2, N1), f32),
            jax.ShapeDtypeStruct((H2, 1), f32),
            jax.ShapeDtypeStruct((H2, 1), f32),
        ],
        interpret=interpret,
    )(y1, W2, g1c, be1c, b2c, s1, q1)

    out = pl.pallas_call(
        _stage3_kernel,
        grid=(B, N1 // BQ2),
        in_specs=[
            pl.BlockSpec((1, H2, BQ2), lambda b, i: (b, 0, i)),
            pl.BlockSpec((H2, 1), lambda b, i: (0, 0)),
            pl.BlockSpec((H2, 1), lambda b, i: (0, 0)),
            pl.BlockSpec((H2, 1), lambda b, i: (0, 0)),
            pl.BlockSpec((H2, 1), lambda b, i: (0, 0)),
        ],
        out_specs=pl.BlockSpec((1, H2, BQ2), lambda b, i: (b, 0, i)),
        out_shape=jax.ShapeDtypeStruct((B, H2, N1), f32),
        interpret=interpret,
    )(y2, g2c, be2c, s2, q2)
    return out


def kernel(xyz1, xyz2, points1, points2, W1, b1, g1, be1, W2, b2, g2, be2):
    return _fp_impl(xyz1, xyz2, points1, points2, W1, b1, g1, be1,
                    W2, b2, g2, be2)


# speedup vs baseline: 2.3429x; 1.0548x over previous
import jax
import jax.numpy as jnp
from jax.experimental import pallas as pl

B, N1, N2, C1, C2 = 4, 4096, 1024, 128, 256
H1, H2 = 256, 128
IN_CH = C1 + C2
BQ = 1024
BQ2 = 2048
NTOT = B * N1
EPS_D = 1e-10
EPS_BN = 1e-5


def _recip(x):
    r = 1.0 / x
    r = r * (2.0 - x * r)
    r = r * (2.0 - x * r)
    return r


def _rsqrt(x):
    y = jax.lax.rsqrt(x)
    y = y * (1.5 - 0.5 * x * y * y)
    y = y * (1.5 - 0.5 * x * y * y)
    return y


def _stage1_kernel(xyz1t_ref, xyz2t_ref, p1_ref, p2_ref, W1_ref, b1_ref,
                   y1_ref, ssum_ref, ssq_ref):
    b = pl.program_id(0)
    i = pl.program_id(1)

    @pl.when(jnp.logical_and(b == 0, i == 0))
    def _init():
        ssum_ref[...] = jnp.zeros_like(ssum_ref)
        ssq_ref[...] = jnp.zeros_like(ssq_ref)

    q = xyz1t_ref[0]
    p = xyz2t_ref[0]
    d = (p[0][:, None] - q[0][None, :]) ** 2
    d = d + (p[1][:, None] - q[1][None, :]) ** 2
    d = d + (p[2][:, None] - q[2][None, :]) ** 2
    d = jnp.maximum(d, EPS_D)
    m1 = jnp.min(d, axis=0)
    d1 = jnp.where(d <= m1[None, :], jnp.inf, d)
    m2 = jnp.min(d1, axis=0)
    d2 = jnp.where(d1 <= m2[None, :], jnp.inf, d1)
    m3 = jnp.min(d2, axis=0)
    w = jnp.where(d <= m3[None, :], 1.0 / d, 0.0)
    wsum = _recip(m1) + _recip(m2) + _recip(m3)
    interp = jnp.dot(p2_ref[0], w, preferred_element_type=jnp.float32)
    interp = interp * _recip(wsum)[None, :]
    x = jnp.concatenate([p1_ref[0], interp], axis=0)
    y1 = jnp.dot(W1_ref[...], x, preferred_element_type=jnp.float32)
    y1 = y1 + b1_ref[...]
    y1_ref[0] = y1
    ssum_ref[...] += jnp.sum(y1, axis=1, keepdims=True)
    ssq_ref[...] += jnp.sum(y1 * y1, axis=1, keepdims=True)


def _stage2_kernel(y1_ref, W2_ref, g1_ref, be1_ref, b2_ref, s_ref, sq_ref,
                   y2_ref, ssum_ref, ssq_ref):
    b = pl.program_id(0)
    i = pl.program_id(1)

    @pl.when(jnp.logical_and(b == 0, i == 0))
    def _init():
        ssum_ref[...] = jnp.zeros_like(ssum_ref)
        ssq_ref[...] = jnp.zeros_like(ssq_ref)

    mean = s_ref[...] * (1.0 / NTOT)
    var = sq_ref[...] * (1.0 / NTOT) - mean * mean
    scale = g1_ref[...] * _rsqrt(var + EPS_BN)
    h = (y1_ref[0] - mean) * scale + be1_ref[...]
    h = jnp.maximum(h, 0.0)
    y2 = jnp.dot(W2_ref[...], h, preferred_element_type=jnp.float32)
    y2 = y2 + b2_ref[...]
    y2_ref[0] = y2
    ssum_ref[...] += jnp.sum(y2, axis=1, keepdims=True)
    ssq_ref[...] += jnp.sum(y2 * y2, axis=1, keepdims=True)


def _stage3_kernel(y2_ref, g2_ref, be2_ref, s_ref, sq_ref, out_ref):
    mean = s_ref[...] * (1.0 / NTOT)
    var = sq_ref[...] * (1.0 / NTOT) - mean * mean
    scale = g2_ref[...] * _rsqrt(var + EPS_BN)
    o = (y2_ref[0] - mean) * scale + be2_ref[...]
    out_ref[0] = jnp.maximum(o, 0.0)


def _fp_impl(xyz1, xyz2, points1, points2, W1, b1, g1, be1, W2, b2, g2, be2,
             interpret=False):
    xyz1t = jnp.transpose(xyz1, (0, 2, 1))
    xyz2t = jnp.transpose(xyz2, (0, 2, 1))
    b1c = b1[:, None]
    g1c = g1[:, None]
    be1c = be1[:, None]
    b2c = b2[:, None]
    g2c = g2[:, None]
    be2c = be2[:, None]

    f32 = jnp.float32
    y1, s1, q1 = pl.pallas_call(
        _stage1_kernel,
        grid=(B, N1 // BQ),
        in_specs=[
            pl.BlockSpec((1, 3, BQ), lambda b, i: (b, 0, i)),
            pl.BlockSpec((1, 3, N2), lambda b, i: (b, 0, 0)),
            pl.BlockSpec((1, C1, BQ), lambda b, i: (b, 0, i)),
            pl.BlockSpec((1, C2, N2), lambda b, i: (b, 0, 0)),
            pl.BlockSpec((H1, IN_CH), lambda b, i: (0, 0)),
            pl.BlockSpec((H1, 1), lambda b, i: (0, 0)),
        ],
        out_specs=[
            pl.BlockSpec((1, H1, BQ), lambda b, i: (b, 0, i)),
            pl.BlockSpec((H1, 1), lambda b, i: (0, 0)),
            pl.BlockSpec((H1, 1), lambda b, i: (0, 0)),
        ],
        out_shape=[
            jax.ShapeDtypeStruct((B, H1, N1), f32),
            jax.ShapeDtypeStruct((H1, 1), f32),
            jax.ShapeDtypeStruct((H1, 1), f32),
        ],
        interpret=interpret,
    )(xyz1t, xyz2t, points1, points2, W1, b1c)

    y2, s2, q2 = pl.pallas_call(
        _stage2_kernel,
        grid=(B, N1 // BQ2),
        in_specs=[
            pl.BlockSpec((1, H1, BQ2), lambda b, i: (b, 0, i)),
            pl.BlockSpec((H2, H1), lambda b, i: (0, 0)),
            pl.BlockSpec((H1, 1), lambda b, i: (0, 0)),
            pl.BlockSpec((H1, 1), lambda b, i: (0, 0)),
            pl.BlockSpec((H2, 1), lambda b, i: (0, 0)),
            pl.BlockSpec((H1, 1), lambda b, i: (0, 0)),
            pl.BlockSpec((H1, 1), lambda b, i: (0, 0)),
        ],
        out_specs=[
            pl.BlockSpec((1, H2, BQ2), lambda b, i: (b, 0, i)),
            pl.BlockSpec((H2, 1), lambda b, i: (0, 0)),
            pl.BlockSpec((H2, 1), lambda b, i: (0, 0)),
        ],
        out_shape=[
            jax.ShapeDtypeStruct((B, H2, N1), f32),
            jax.ShapeDtypeStruct((H2, 1), f32),
            jax.ShapeDtypeStruct((H2, 1), f32),
        ],
        interpret=interpret,
    )(y1, W2, g1c, be1c, b2c, s1, q1)

    out = pl.pallas_call(
        _stage3_kernel,
        grid=(B, N1 // BQ2),
        in_specs=[
            pl.BlockSpec((1, H2, BQ2), lambda b, i: (b, 0, i)),
            pl.BlockSpec((H2, 1), lambda b, i: (0, 0)),
            pl.BlockSpec((H2, 1), lambda b, i: (0, 0)),
            pl.BlockSpec((H2, 1), lambda b, i: (0, 0)),
            pl.BlockSpec((H2, 1), lambda b, i: (0, 0)),
        ],
        out_specs=pl.BlockSpec((1, H2, BQ2), lambda b, i: (b, 0, i)),
        out_shape=jax.ShapeDtypeStruct((B, H2, N1), f32),
        interpret=interpret,
    )(y2, g2c, be2c, s2, q2)
    return out


def kernel(xyz1, xyz2, points1, points2, W1, b1, g1, be1, W2, b2, g2, be2):
    return _fp_impl(xyz1, xyz2, points1, points2, W1, b1, g1, be1,
                    W2, b2, g2, be2)
